# Initial kernel scaffold; baseline (speedup 1.0000x reference)
import jax
import jax.numpy as jnp
from jax.experimental import pallas as pl


def kernel(box_prompts, num_boxes):
    raise NotImplementedError("write your pallas kernel here")


# speedup vs baseline: 9.6095x; 9.6095x over previous
import jax
import jax.numpy as jnp
from jax import lax
from jax.experimental import pallas as pl
from jax.experimental.pallas import tpu as pltpu

_THR = 0.8
_N = 1024


def _filter_kernel(raw_ref, tr_ref, num_ref, out_ref, nk_ref):
    n = num_ref[0, 0, 0]
    raw = raw_ref[0]
    tr = tr_ref[0]
    x1r, y1r, x2r, y2r = tr[0:1, :], tr[1:2, :], tr[2:3, :], tr[3:4, :]
    x1c, y1c, x2c, y2c = raw[:, 0:1], raw[:, 1:2], raw[:, 2:3], raw[:, 3:4]
    area_r = (x2r - x1r) * (y2r - y1r)
    area_c = (x2c - x1c) * (y2c - y1c)
    ir = lax.broadcasted_iota(jnp.int32, (1, _N), 1)
    ic = lax.broadcasted_iota(jnp.int32, (_N, 1), 0)
    vr = ir < n
    vc = ic < n
    d = (x1c >= x1r) & (y1c >= y1r) & (x2c <= x2r) & (y2c <= y2r)
    d = d & vr & vc & (ic != ir)
    w = jnp.where(d, jnp.broadcast_to(area_c, (_N, _N)), 0.0)
    sum_contained = jnp.sum(w, axis=0, keepdims=True)
    keep = (sum_contained <= _THR * (area_r + 1e-9)) & vr
    keep_f = keep.astype(jnp.float32)
    le = (ic <= ir).astype(jnp.float32)
    cum = jnp.dot(keep_f, le, preferred_element_type=jnp.float32)
    pos = cum - 1.0
    p_mat = jnp.where((pos == ic.astype(jnp.float32)) & keep, 1.0, 0.0)
    filt = jnp.dot(p_mat, raw, preferred_element_type=jnp.float32)
    nk = jnp.sum(keep_f).astype(jnp.int32)
    out_ref[0] = jnp.where(nk > 0, filt, raw)
    nk_ref[0, 0, 0] = nk


def kernel(box_prompts, num_boxes):
    T, C, MAXB, F = box_prompts.shape
    S = T * C
    raw = box_prompts.reshape(S, MAXB, F)
    raw = jnp.pad(raw, ((0, 0), (0, _N - MAXB), (0, 0)))
    tr = raw.transpose(0, 2, 1)
    num = num_boxes.reshape(S, 1, 1)
    out, nk = pl.pallas_call(
        _filter_kernel,
        grid=(S,),
        in_specs=[
            pl.BlockSpec((1, _N, F), lambda i: (i, 0, 0)),
            pl.BlockSpec((1, F, _N), lambda i: (i, 0, 0)),
            pl.BlockSpec((1, 1, 1), lambda i: (i, 0, 0), memory_space=pltpu.SMEM),
        ],
        out_specs=[
            pl.BlockSpec((1, _N, F), lambda i: (i, 0, 0)),
            pl.BlockSpec((1, 1, 1), lambda i: (i, 0, 0), memory_space=pltpu.SMEM),
        ],
        out_shape=[
            jax.ShapeDtypeStruct((S, _N, F), jnp.float32),
            jax.ShapeDtypeStruct((S, 1, 1), jnp.int32),
        ],
        compiler_params=pltpu.CompilerParams(
            dimension_semantics=("parallel",)
        ),
    )(raw, tr, num)
    filtered = out[:, :MAXB, :].reshape(T, C, MAXB, F)
    return filtered, nk.reshape(T, C)
